# Initial kernel scaffold; baseline (speedup 1.0000x reference)
import jax
import jax.numpy as jnp
from jax.experimental import pallas as pl


def kernel(x, edge_index, Ws, bs, att_vec):
    raise NotImplementedError("write your pallas kernel here")


# speedup vs baseline: 7.8921x; 7.8921x over previous
import functools

import jax
import jax.numpy as jnp
from jax import lax
from jax.experimental import pallas as pl
from jax.experimental.pallas import tpu as pltpu
from jax.experimental.pallas import tpu_sc as plsc

N = 10000
E = 320000
D = 128
NC = 2
NS = 16
NW = NC * NS
EPW = E // NW
CH = 80
NCH = EPW // CH
SEG = 640
NP = NS * SEG

_mesh = plsc.VectorSubcoreMesh(core_axis_name="c", subcore_axis_name="s")

_f32 = jnp.float32
_i32 = jnp.int32


def _wid():
    return lax.axis_index("s") * NC + lax.axis_index("c")


@functools.partial(
    pl.kernel,
    out_type=jax.ShapeDtypeStruct((NC * NP,), _f32),
    mesh=_mesh,
    compiler_params=pltpu.CompilerParams(needs_layout_passes=False),
    scratch_types=[
        pltpu.VMEM((NP,), _f32),
        pltpu.VMEM((EPW,), _i32),
        pltpu.VMEM((EPW,), _i32),
        pltpu.VMEM((NS * SEG,), _f32),
        pltpu.VMEM((SEG,), _f32),
        pltpu.VMEM_SHARED((NS * NP,), _f32),
    ],
)
def _deg_kernel(src_hbm, dst_hbm, degpart_hbm, deg_l, src_v, dst_v,
                rbuf, obuf, shared):
    c = lax.axis_index("c")
    t = lax.axis_index("s")
    wid = _wid()
    z16 = jnp.zeros((16,), _f32)

    def _zero(j, _):
        deg_l[pl.ds(j * 16, 16)] = z16
        return 0
    lax.fori_loop(0, NP // 16, _zero, 0)

    base_e = wid * EPW
    pltpu.sync_copy(src_hbm.at[pl.ds(base_e, EPW)], src_v)
    pltpu.sync_copy(dst_hbm.at[pl.ds(base_e, EPW)], dst_v)

    lanes = lax.iota(_i32, 16)
    ones = jnp.ones((16,), _f32)

    def _scan(j, _):
        sl = pl.ds(j * 16, 16)
        sv = src_v[sl]
        dv = dst_v[sl]
        ns = sv != dv
        for lane in range(16):
            plsc.addupdate_scatter(deg_l, [dv], ones,
                                   mask=ns & (lanes == lane))
        return 0
    lax.fori_loop(0, EPW // 16, _scan, 0)

    pltpu.sync_copy(deg_l, shared.at[pl.ds(t * NP, NP)])
    plsc.subcore_barrier()

    base_n = t * SEG
    for k in range(NS):
        pltpu.sync_copy(shared.at[pl.ds(k * NP + base_n, SEG)],
                        rbuf.at[pl.ds(k * SEG, SEG)])

    def _red(j, _):
        acc = rbuf[pl.ds(j * 16, 16)]
        for k in range(1, NS):
            acc = acc + rbuf[pl.ds(k * SEG + j * 16, 16)]
        obuf[pl.ds(j * 16, 16)] = acc
        return 0
    lax.fori_loop(0, SEG // 16, _red, 0)
    pltpu.sync_copy(obuf, degpart_hbm.at[pl.ds(c * NP + base_n, SEG)])


@functools.partial(
    pl.kernel,
    out_type=jax.ShapeDtypeStruct((E,), _f32),
    mesh=_mesh,
    compiler_params=pltpu.CompilerParams(needs_layout_passes=False),
    scratch_types=[
        pltpu.VMEM((NP,), _f32),
        pltpu.VMEM((NP,), _f32),
        pltpu.VMEM((NP,), _f32),
        pltpu.VMEM((EPW,), _i32),
        pltpu.VMEM((EPW,), _i32),
        pltpu.VMEM((EPW,), _f32),
    ],
)
def _weight_kernel(src_hbm, dst_hbm, degpart_hbm, w_hbm,
                   a_v, b_v, dinv_v, src_v, dst_v, w_v):
    wid = _wid()
    pltpu.sync_copy(degpart_hbm.at[pl.ds(0, NP)], a_v)
    pltpu.sync_copy(degpart_hbm.at[pl.ds(NP, NP)], b_v)

    def _rsqrt(j, _):
        sl = pl.ds(j * 16, 16)
        d = a_v[sl] + b_v[sl]
        i = plsc.bitcast(d, _i32)
        i = _i32(0x5F3759DF) - jnp.right_shift(i, 1)
        y = plsc.bitcast(i, _f32)
        hd = 0.5 * d
        for _ in range(3):
            y = y * (1.5 - hd * y * y)
        dinv_v[sl] = jnp.where(d > 0.0, y, 0.0)
        return 0
    lax.fori_loop(0, NP // 16, _rsqrt, 0)

    base_e = wid * EPW
    pltpu.sync_copy(src_hbm.at[pl.ds(base_e, EPW)], src_v)
    pltpu.sync_copy(dst_hbm.at[pl.ds(base_e, EPW)], dst_v)

    def _wloop(j, _):
        sl = pl.ds(j * 16, 16)
        sv = src_v[sl]
        dv = dst_v[sl]
        ds_ = plsc.load_gather(dinv_v, [sv])
        dd_ = plsc.load_gather(dinv_v, [dv])
        w_v[sl] = jnp.where(sv != dv, 0.5 * ds_ * dd_, 0.0)
        return 0
    lax.fori_loop(0, EPW // 16, _wloop, 0)
    pltpu.sync_copy(w_v, w_hbm.at[pl.ds(base_e, EPW)])


NH = N // NC
EPT = E // NS
NCHT = EPT // CH
SB = 50
NSB = NCHT // SB
TSEG = 320


@functools.partial(
    pl.kernel,
    out_type=jax.ShapeDtypeStruct((N, D), _f32),
    mesh=_mesh,
    compiler_params=pltpu.CompilerParams(needs_layout_passes=False),
    scratch_types=[
        pltpu.VMEM((SB, CH), _i32),
        pltpu.VMEM((SB, CH), _i32),
        pltpu.VMEM((SB, CH), _f32),
        pltpu.VMEM((CH,), _i32),
        pltpu.VMEM((CH, D), _f32),
        pltpu.VMEM((CH, D), _f32),
        pltpu.VMEM((64, D), _f32),
        pltpu.SemaphoreType.DMA,
        pltpu.SemaphoreType.DMA,
        pltpu.VMEM_SHARED((NH, D), _f32),
    ],
)
def _hop_kernel(h_hbm, src_hbm, dst_hbm, w_hbm, out_hbm,
                src_v, dst_v, w_v, loc_v, rows_a, rows_b, ibuf,
                sem_a, sem_b, acc):
    c = lax.axis_index("c")
    t = lax.axis_index("s")
    nbase = c * NH

    row0 = jnp.minimum(t * TSEG, NH - TSEG)

    def _initc(k, _):
        b = row0 + k * 64
        pltpu.sync_copy(h_hbm.at[pl.ds(nbase + b, 64)], ibuf)

        def _scale(r, _):
            for j in range(D // 16):
                sl = pl.ds(j * 16, 16)
                ibuf[r, sl] = ibuf[r, sl] * 0.5
            return 0
        lax.fori_loop(0, 64, _scale, 0)
        pltpu.sync_copy(ibuf, acc.at[pl.ds(b, 64)])
        return 0
    lax.fori_loop(0, TSEG // 64, _initc, 0)

    plsc.subcore_barrier()

    def _gather(cidx, buf, sem):
        pltpu.async_copy(h_hbm.at[src_v.at[cidx]], buf, sem)

    def _wait(buf, sem):
        pltpu.make_async_copy(h_hbm.at[src_v.at[0]], buf, sem).wait()

    def _process(cidx, buf):
        def _srow(bb, _):
            sl16 = pl.ds(bb * 16, 16)
            w16 = w_v[cidx, sl16]
            d16 = dst_v[cidx, sl16]
            inr = (d16 >= nbase) & (d16 < nbase + NH)
            we = jnp.where(inr, w16, 0.0)
            loc_v[sl16] = jnp.clip(d16 - nbase, 0, NH - 1)
            for lane in range(16):
                wv = jnp.full((16,), we[lane], dtype=_f32)
                b = bb * 16 + lane
                for j in range(D // 16):
                    sl = pl.ds(j * 16, 16)
                    buf[b, sl] = buf[b, sl] * wv
            return 0
        lax.fori_loop(0, CH // 16, _srow, 0)
        pltpu.sync_copy(buf, acc.at[loc_v], add=True)

    def _super(sb, _):
        pltpu.sync_copy(src_hbm.at[t, sb], src_v)
        pltpu.sync_copy(dst_hbm.at[t, sb], dst_v)
        pltpu.sync_copy(w_hbm.at[t, sb], w_v)

        _gather(0, rows_a, sem_a)

        def _pair(g, _):
            c0 = 2 * g
            _wait(rows_a, sem_a)
            _gather(c0 + 1, rows_b, sem_b)
            _process(c0, rows_a)
            _wait(rows_b, sem_b)
            _gather(c0 + 2, rows_a, sem_a)
            _process(c0 + 1, rows_b)
            return 0
        lax.fori_loop(0, (SB - 1) // 2, _pair, 0)

        if SB % 2 == 0:
            _wait(rows_a, sem_a)
            _gather(SB - 1, rows_b, sem_b)
            _process(SB - 2, rows_a)
            _wait(rows_b, sem_b)
            _process(SB - 1, rows_b)
        else:
            _wait(rows_a, sem_a)
            _process(SB - 1, rows_a)
        return 0
    lax.fori_loop(0, NSB, _super, 0)

    plsc.subcore_barrier()
    pltpu.sync_copy(acc.at[pl.ds(row0, TSEG)],
                    out_hbm.at[pl.ds(nbase + row0, TSEG)])


_BN = 400


def _dense_body(p0_ref, p1_ref, p2_ref, p4_ref,
                ws_ref, bs_ref, av_ref, out_ref):
    p0 = p0_ref[...]
    p1 = p1_ref[...]
    p2 = p2_ref[...]
    p4 = p4_ref[...]
    hs = (p0, p1, p2, p4,
          jnp.abs(p0 - p1), jnp.abs(p1 - p2), jnp.abs(p2 - p4))
    zs = []
    ss = []
    for i in range(7):
        z = jnp.dot(hs[i], ws_ref[i], preferred_element_type=_f32)
        z = z + bs_ref[i][None, :]
        r = jnp.maximum(z, 0.0)
        s = jnp.dot(r, av_ref[i][:, None], preferred_element_type=_f32)
        ss.append(jnp.maximum(s, 0.0))
        zs.append(z)
    m = ss[0]
    for i in range(1, 7):
        m = jnp.maximum(m, ss[i])
    es = [jnp.exp(s - m) for s in ss]
    den = es[0]
    for i in range(1, 7):
        den = den + es[i]
    acc = es[0] * zs[0]
    for i in range(1, 7):
        acc = acc + es[i] * zs[i]
    o = acc / den
    out_ref[...] = jnp.where(o > 0, o, 0.01 * o)


def _dense():
    n_ch = 7
    grid = (N // _BN,)
    blk = pl.BlockSpec((_BN, D), lambda i: (i, 0))
    return pl.pallas_call(
        _dense_body,
        grid=grid,
        in_specs=[
            blk, blk, blk, blk,
            pl.BlockSpec((n_ch, D, D), lambda i: (0, 0, 0)),
            pl.BlockSpec((n_ch, D), lambda i: (0, 0)),
            pl.BlockSpec((n_ch, D), lambda i: (0, 0)),
        ],
        out_specs=blk,
        out_shape=jax.ShapeDtypeStruct((N, D), _f32),
    )


def kernel(x, edge_index, Ws, bs, att_vec):
    src = edge_index[0].astype(_i32)
    dst = edge_index[1].astype(_i32)
    degpart = _deg_kernel(src, dst)
    w = _weight_kernel(src, dst, degpart)
    src3 = src.reshape(NS, NSB, SB, CH)
    dst3 = dst.reshape(NS, NSB, SB, CH)
    w3 = w.reshape(NS, NSB, SB, CH)

    p1 = _hop_kernel(x, src3, dst3, w3)
    p2 = _hop_kernel(p1, src3, dst3, w3)
    p3 = _hop_kernel(p2, src3, dst3, w3)
    p4 = _hop_kernel(p3, src3, dst3, w3)

    return _dense()(x, p1, p2, p4, Ws, bs, att_vec)
